# initial kernel scaffold (unmeasured)
import jax
import jax.numpy as jnp
from jax import lax
from jax.experimental import pallas as pl
from jax.experimental.pallas import tpu as pltpu


def kernel(
    x,
):
    def body(*refs):
        pass

    out_shape = jax.ShapeDtypeStruct(..., jnp.float32)
    return pl.pallas_call(body, out_shape=out_shape)(...)


# baseline (device time: 132548 ns/iter reference)
import jax
import jax.numpy as jnp
from jax import lax
from jax.experimental import pallas as pl
from jax.experimental.pallas import tpu as pltpu

P = 32


def kernel(x):
    m, n = x.shape
    c = m // P

    def body(x_ref, out_ref, first_ref, rs_ref, ag_ref,
             rs_send_sems, rs_recv_sems, ag_send_sems, ag_recv_sems):
        my = lax.axis_index("i")
        left = lax.rem(my + (P - 1), P)
        right = lax.rem(my + 1, P)

        barrier_sem = pltpu.get_barrier_semaphore()
        for nbr in (left, right):
            pl.semaphore_signal(
                barrier_sem, inc=1,
                device_id=(nbr,), device_id_type=pl.DeviceIdType.MESH,
            )
        pl.semaphore_wait(barrier_sem, 2)

        first_ref[...] = x_ref[pl.ds(my * c, c), :].astype(jnp.bfloat16)

        for h in range(P - 1):
            src = first_ref if h == 0 else rs_ref.at[h - 1]
            rdma = pltpu.make_async_remote_copy(
                src_ref=src,
                dst_ref=rs_ref.at[h],
                send_sem=rs_send_sems.at[h],
                recv_sem=rs_recv_sems.at[h],
                device_id=(right,),
                device_id_type=pl.DeviceIdType.MESH,
            )
            rdma.start()
            rdma.wait()
            ci = lax.rem(my - (h + 1) + 2 * P, P)
            rs_ref[h, :, :] = (
                rs_ref[h, :, :]
                + x_ref[pl.ds(ci * c, c), :].astype(jnp.bfloat16)
            )

        r_idx = lax.rem(my + 1, P)
        out_ref[pl.ds(r_idx * c, c), :] = rs_ref[P - 2].astype(jnp.float32)

        for h in range(P - 1):
            src = rs_ref.at[P - 2] if h == 0 else ag_ref.at[h - 1]
            rdma = pltpu.make_async_remote_copy(
                src_ref=src,
                dst_ref=ag_ref.at[h],
                send_sem=ag_send_sems.at[h],
                recv_sem=ag_recv_sems.at[h],
                device_id=(right,),
                device_id_type=pl.DeviceIdType.MESH,
            )
            rdma.start()
            rdma.wait()
            ci = lax.rem(my - h + 2 * P, P)
            out_ref[pl.ds(ci * c, c), :] = ag_ref[h].astype(jnp.float32)

    return pl.pallas_call(
        body,
        out_shape=jax.ShapeDtypeStruct((m, n), jnp.float32),
        in_specs=[pl.BlockSpec(memory_space=pltpu.VMEM)],
        out_specs=pl.BlockSpec(memory_space=pltpu.VMEM),
        scratch_shapes=[
            pltpu.VMEM((c, n), jnp.bfloat16),
            pltpu.VMEM((P - 1, c, n), jnp.bfloat16),
            pltpu.VMEM((P - 1, c, n), jnp.bfloat16),
            pltpu.SemaphoreType.DMA((P - 1,)),
            pltpu.SemaphoreType.DMA((P - 1,)),
            pltpu.SemaphoreType.DMA((P - 1,)),
            pltpu.SemaphoreType.DMA((P - 1,)),
        ],
        compiler_params=pltpu.CompilerParams(collective_id=0),
    )(x)


# device time: 23379 ns/iter; 5.6695x vs baseline; 5.6695x over previous
import jax
import jax.numpy as jnp
from jax import lax
from jax.experimental import pallas as pl
from jax.experimental.pallas import tpu as pltpu

P = 32


def kernel(x):
    m, n = x.shape
    c = m // P

    def body(x_ref, out_ref, stage_ref, rs_ref, red_ref, ag_ref,
             rs_send_sems, rs_recv_sems, ag_send_sems, ag_recv_sems):
        my = lax.axis_index("i")

        barrier_sem = pltpu.get_barrier_semaphore()
        for o in range(1, P):
            peer = lax.rem(my + o, P)
            pl.semaphore_signal(
                barrier_sem, inc=1,
                device_id=(peer,), device_id_type=pl.DeviceIdType.MESH,
            )
        pl.semaphore_wait(barrier_sem, P - 1)

        for o in range(1, P):
            e = lax.rem(my + o, P)
            stage_ref[o - 1, :, :] = (
                x_ref[pl.ds(e * c, c), :].astype(jnp.bfloat16)
            )

        rs_sends = []
        for o in range(1, P):
            e = lax.rem(my + o, P)
            rdma = pltpu.make_async_remote_copy(
                src_ref=stage_ref.at[o - 1],
                dst_ref=rs_ref.at[o - 1],
                send_sem=rs_send_sems.at[o - 1],
                recv_sem=rs_recv_sems.at[o - 1],
                device_id=(e,),
                device_id_type=pl.DeviceIdType.MESH,
            )
            rdma.start()
            rs_sends.append(rdma)

        red = x_ref[pl.ds(my * c, c), :]
        for o in range(1, P):
            rs_sends[o - 1].wait_recv()
            red = red + rs_ref[o - 1, :, :].astype(jnp.float32)
        red_ref[...] = red.astype(jnp.bfloat16)
        out_ref[pl.ds(my * c, c), :] = red_ref[...]

        ag_sends = []
        for o in range(1, P):
            e = lax.rem(my + o, P)
            rdma = pltpu.make_async_remote_copy(
                src_ref=red_ref,
                dst_ref=ag_ref.at[o - 1],
                send_sem=ag_send_sems.at[o - 1],
                recv_sem=ag_recv_sems.at[o - 1],
                device_id=(e,),
                device_id_type=pl.DeviceIdType.MESH,
            )
            rdma.start()
            ag_sends.append(rdma)

        for r in rs_sends:
            r.wait_send()

        for o in range(1, P):
            ag_sends[o - 1].wait_recv()
            s = lax.rem(my - o + P, P)
            out_ref[pl.ds(s * c, c), :] = ag_ref[o - 1, :, :]

        for r in ag_sends:
            r.wait_send()

    return pl.pallas_call(
        body,
        out_shape=jax.ShapeDtypeStruct((m, n), jnp.bfloat16),
        in_specs=[pl.BlockSpec(memory_space=pltpu.VMEM)],
        out_specs=pl.BlockSpec(memory_space=pltpu.VMEM),
        scratch_shapes=[
            pltpu.VMEM((P - 1, c, n), jnp.bfloat16),
            pltpu.VMEM((P - 1, c, n), jnp.bfloat16),
            pltpu.VMEM((c, n), jnp.bfloat16),
            pltpu.VMEM((P - 1, c, n), jnp.bfloat16),
            pltpu.SemaphoreType.DMA((P - 1,)),
            pltpu.SemaphoreType.DMA((P - 1,)),
            pltpu.SemaphoreType.DMA((P - 1,)),
            pltpu.SemaphoreType.DMA((P - 1,)),
        ],
        compiler_params=pltpu.CompilerParams(collective_id=0),
    )(x)
